# Initial kernel scaffold; baseline (speedup 1.0000x reference)
import jax
import jax.numpy as jnp
from jax.experimental import pallas as pl


def kernel(x, W):
    raise NotImplementedError("write your pallas kernel here")


# speedup vs baseline: 1.0610x; 1.0610x over previous
import functools

import jax
import jax.numpy as jnp
from jax.experimental import pallas as pl

_N = 16384
_H = 4096
_E = 64
_K = 8
_ROWS = 512


def _router_body(x_ref, w_ref, idx_ref, wgt_ref, probs_ref):
    logits = jnp.dot(x_ref[...], w_ref[...], preferred_element_type=jnp.float32)
    m = jnp.max(logits, axis=-1, keepdims=True)
    e = jnp.exp(logits - m)
    s = jnp.sum(e, axis=-1, keepdims=True)
    probs = e / s
    probs_ref[...] = probs

    rows = probs.shape[0]
    cols = jax.lax.broadcasted_iota(jnp.int32, (rows, _E), 1)
    work = probs
    top_v = []
    top_i = []
    for _ in range(_K):
        mx = jnp.max(work, axis=-1, keepdims=True)
        idx = jnp.min(jnp.where(work == mx, cols, _E), axis=-1, keepdims=True)
        top_v.append(mx)
        top_i.append(idx)
        work = jnp.where(cols == idx, -1.0, work)
    vals = jnp.concatenate(top_v, axis=-1)
    idxs = jnp.concatenate(top_i, axis=-1)
    wgt_ref[...] = vals / jnp.sum(vals, axis=-1, keepdims=True)
    idx_ref[...] = idxs


@functools.partial(jax.jit, static_argnames=())
def kernel(x, W):
    n, h = x.shape
    e = W.shape[1]
    rows = _ROWS
    grid = (n // rows,)
    out_shapes = (
        jax.ShapeDtypeStruct((n, _K), jnp.int32),
        jax.ShapeDtypeStruct((n, _K), jnp.float32),
        jax.ShapeDtypeStruct((n, e), jnp.float32),
    )
    return pl.pallas_call(
        _router_body,
        grid=grid,
        in_specs=[
            pl.BlockSpec((rows, h), lambda i: (i, 0)),
            pl.BlockSpec((h, e), lambda i: (0, 0)),
        ],
        out_specs=(
            pl.BlockSpec((rows, _K), lambda i: (i, 0)),
            pl.BlockSpec((rows, _K), lambda i: (i, 0)),
            pl.BlockSpec((rows, e), lambda i: (i, 0)),
        ),
        out_shape=out_shapes,
    )(x, W)


# speedup vs baseline: 1.1550x; 1.0886x over previous
import functools

import jax
import jax.numpy as jnp
from jax.experimental import pallas as pl

_N = 16384
_H = 4096
_E = 64
_K = 8
_ROWS = 512


def _router_body(x_ref, w_ref, idx_ref, wgt_ref, probs_ref):
    logits = jnp.dot(x_ref[...], w_ref[...], preferred_element_type=jnp.float32)
    m = jnp.max(logits, axis=-1, keepdims=True)
    e = jnp.exp(logits - m)
    s = jnp.sum(e, axis=-1, keepdims=True)
    probs = e / s
    probs_ref[...] = probs

    rows = probs.shape[0]
    cols = jax.lax.broadcasted_iota(jnp.int32, (rows, _E), 1)
    bits = jax.lax.bitcast_convert_type(probs, jnp.int32)
    work = (bits & ~(_E - 1)) | ((_E - 1) - cols)
    top_keys = []
    for _ in range(_K):
        mx = jnp.max(work, axis=-1, keepdims=True)
        top_keys.append(mx)
        work = jnp.where(work == mx, -1, work)
    keys = jnp.concatenate(top_keys, axis=-1)
    idxs = (_E - 1) - (keys & (_E - 1))
    vals = jax.lax.bitcast_convert_type(keys & ~(_E - 1), jnp.float32)
    wgt_ref[...] = vals / jnp.sum(vals, axis=-1, keepdims=True)
    idx_ref[...] = idxs


@functools.partial(jax.jit, static_argnames=())
def kernel(x, W):
    n, h = x.shape
    e = W.shape[1]
    rows = _ROWS
    grid = (n // rows,)
    out_shapes = (
        jax.ShapeDtypeStruct((n, _K), jnp.int32),
        jax.ShapeDtypeStruct((n, _K), jnp.float32),
        jax.ShapeDtypeStruct((n, e), jnp.float32),
    )
    return pl.pallas_call(
        _router_body,
        grid=grid,
        in_specs=[
            pl.BlockSpec((rows, h), lambda i: (i, 0)),
            pl.BlockSpec((h, e), lambda i: (0, 0)),
        ],
        out_specs=(
            pl.BlockSpec((rows, _K), lambda i: (i, 0)),
            pl.BlockSpec((rows, _K), lambda i: (i, 0)),
            pl.BlockSpec((rows, e), lambda i: (i, 0)),
        ),
        out_shape=out_shapes,
    )(x, W)


# speedup vs baseline: 1.2594x; 1.0904x over previous
import functools

import jax
import jax.numpy as jnp
from jax.experimental import pallas as pl

_N = 16384
_H = 4096
_E = 64
_K = 8
_ROWS = 512


def _router_body(x_ref, w_ref, idx_ref, wgt_ref, probs_ref):
    logits = jnp.dot(x_ref[...], w_ref[...], preferred_element_type=jnp.float32)
    m = jnp.max(logits, axis=-1, keepdims=True)
    e = jnp.exp(logits - m)
    s = jnp.sum(e, axis=-1, keepdims=True)
    probs = e / s
    probs_ref[...] = probs

    rows = probs.shape[0]
    cols = jax.lax.broadcasted_iota(jnp.int32, (rows, _E), 1)
    bits = jax.lax.bitcast_convert_type(probs, jnp.int32)
    work = jax.lax.bitcast_convert_type(
        (bits & ~(_E - 1)) | ((_E - 1) - cols), jnp.float32)
    top_keys = []
    for _ in range(_K):
        mx = jnp.max(work, axis=-1, keepdims=True)
        top_keys.append(mx)
        work = jnp.where(work == mx, -1.0, work)
    keys = jax.lax.bitcast_convert_type(
        jnp.concatenate(top_keys, axis=-1), jnp.int32)
    idxs = (_E - 1) - (keys & (_E - 1))
    vals = jax.lax.bitcast_convert_type(keys & ~(_E - 1), jnp.float32)
    wgt_ref[...] = vals / jnp.sum(vals, axis=-1, keepdims=True)
    idx_ref[...] = idxs


@functools.partial(jax.jit, static_argnames=())
def kernel(x, W):
    n, h = x.shape
    e = W.shape[1]
    rows = _ROWS
    grid = (n // rows,)
    out_shapes = (
        jax.ShapeDtypeStruct((n, _K), jnp.int32),
        jax.ShapeDtypeStruct((n, _K), jnp.float32),
        jax.ShapeDtypeStruct((n, e), jnp.float32),
    )
    return pl.pallas_call(
        _router_body,
        grid=grid,
        in_specs=[
            pl.BlockSpec((rows, h), lambda i: (i, 0)),
            pl.BlockSpec((h, e), lambda i: (0, 0)),
        ],
        out_specs=(
            pl.BlockSpec((rows, _K), lambda i: (i, 0)),
            pl.BlockSpec((rows, _K), lambda i: (i, 0)),
            pl.BlockSpec((rows, e), lambda i: (i, 0)),
        ),
        out_shape=out_shapes,
    )(x, W)


# speedup vs baseline: 1.3624x; 1.0818x over previous
import functools

import jax
import jax.numpy as jnp
from jax.experimental import pallas as pl

_N = 16384
_H = 4096
_E = 64
_K = 8
_ROWS = 1024


def _router_body(x_ref, w_ref, idx_ref, wgt_ref, probs_ref):
    logits = jnp.dot(x_ref[...], w_ref[...], preferred_element_type=jnp.float32)
    m = jnp.max(logits, axis=-1, keepdims=True)
    e = jnp.exp(logits - m)
    s = jnp.sum(e, axis=-1, keepdims=True)
    probs = e / s
    probs_ref[...] = probs

    rows = probs.shape[0]
    cols = jax.lax.broadcasted_iota(jnp.int32, (rows, _E), 1)
    bits = jax.lax.bitcast_convert_type(probs, jnp.int32)
    work = jax.lax.bitcast_convert_type(
        (bits & ~(_E - 1)) | ((_E - 1) - cols), jnp.float32)
    top_keys = []
    for _ in range(_K):
        mx = jnp.max(work, axis=-1, keepdims=True)
        top_keys.append(mx)
        work = jnp.where(work == mx, -1.0, work)
    keys = jax.lax.bitcast_convert_type(
        jnp.concatenate(top_keys, axis=-1), jnp.int32)
    idxs = (_E - 1) - (keys & (_E - 1))
    vals = jax.lax.bitcast_convert_type(keys & ~(_E - 1), jnp.float32)
    wgt_ref[...] = vals / jnp.sum(vals, axis=-1, keepdims=True)
    idx_ref[...] = idxs


@functools.partial(jax.jit, static_argnames=())
def kernel(x, W):
    n, h = x.shape
    e = W.shape[1]
    rows = _ROWS
    grid = (n // rows,)
    out_shapes = (
        jax.ShapeDtypeStruct((n, _K), jnp.int32),
        jax.ShapeDtypeStruct((n, _K), jnp.float32),
        jax.ShapeDtypeStruct((n, e), jnp.float32),
    )
    return pl.pallas_call(
        _router_body,
        grid=grid,
        in_specs=[
            pl.BlockSpec((rows, h), lambda i: (i, 0)),
            pl.BlockSpec((h, e), lambda i: (0, 0)),
        ],
        out_specs=(
            pl.BlockSpec((rows, _K), lambda i: (i, 0)),
            pl.BlockSpec((rows, _K), lambda i: (i, 0)),
            pl.BlockSpec((rows, e), lambda i: (i, 0)),
        ),
        out_shape=out_shapes,
    )(x, W)
